# baseline (speedup 1.0000x reference)
import jax
import jax.numpy as jnp
from jax import lax
from jax.experimental import pallas as pl
from jax.experimental.pallas import tpu as pltpu
from jax.experimental.pallas import tpu_sc as plsc

B, N, K, E = 4, 2500, 12, 7500
NW = 32
WPB = NW // B
NP = 2560
CH = NP // WPB
EP = 7680
GROUPS = CH // 16


def _sc_loss(oe_hbm, ve_hbm, vs_hbm, gt_hbm, out_hbm,
             oe_v, ve_v, vs_v, gt_v, loss_v, sem):
    c = lax.axis_index("c")
    s = lax.axis_index("s")
    wid = s * 2 + c
    batch = wid // WPB
    sub = lax.rem(wid, WPB)

    cp0 = pltpu.async_copy(oe_hbm.at[batch], oe_v, sem)
    cp1 = pltpu.async_copy(ve_hbm.at[batch, sub], ve_v, sem)
    cp2 = pltpu.async_copy(vs_hbm.at[batch, sub], vs_v, sem)
    cp3 = pltpu.async_copy(gt_hbm.at[batch, sub], gt_v, sem)
    cp0.wait()
    cp1.wait()
    cp2.wait()
    cp3.wait()

    iota = lax.iota(jnp.int32, 16)
    iK = iota * K
    i3 = iota * 3

    def group(g, acc):
        jb = g * 16
        osum = None
        for k in range(K):
            pos = iK + (jb * K + k)
            vidx = plsc.load_gather(ve_v, [pos])
            val = plsc.load_gather(oe_v, [vidx])
            osum = val if osum is None else osum + val
        off = osum * (1.0 / K)
        d = []
        for ccoord in range(3):
            pos = i3 + (jb * 3 + ccoord)
            d.append(plsc.load_gather(vs_v, [pos]) -
                     plsc.load_gather(gt_v, [pos]))
        d0, d1, d2 = d
        a = (d0 * d0 + d1 * d1 + d2 * d2) * (1.0 / 3.0)
        bd = (d0 + d1 + d2) * (2.0 / 3.0)
        return acc + a + off * (bd + off)

    loss16 = lax.fori_loop(0, GROUPS, group, jnp.zeros((16,), jnp.float32))
    loss_v[...] = loss16
    pltpu.sync_copy(loss_v, out_hbm.at[wid])


@jax.jit
def kernel(out_edges, gt_vs, vs, ve):
    oe = jnp.pad(out_edges[:, 0, :], ((0, 0), (0, EP - E)))
    ve_p = jnp.pad(ve, ((0, 0), (0, NP - N), (0, 0)),
                   constant_values=E).reshape(B, WPB, CH * K)
    vs_p = jnp.pad(vs, ((0, 0), (0, NP - N), (0, 0))).reshape(B, WPB, CH * 3)
    gt_p = jnp.pad(gt_vs, ((0, 0), (0, NP - N), (0, 0))).reshape(B, WPB, CH * 3)

    mesh = plsc.VectorSubcoreMesh(core_axis_name="c", subcore_axis_name="s")
    run = pl.kernel(
        _sc_loss,
        out_type=jax.ShapeDtypeStruct((NW, 16), jnp.float32),
        mesh=mesh,
        compiler_params=pltpu.CompilerParams(needs_layout_passes=False),
        scratch_types=[
            pltpu.VMEM((EP,), jnp.float32),
            pltpu.VMEM((CH * K,), jnp.int32),
            pltpu.VMEM((CH * 3,), jnp.float32),
            pltpu.VMEM((CH * 3,), jnp.float32),
            pltpu.VMEM((16,), jnp.float32),
            pltpu.SemaphoreType.DMA,
        ],
    )
    partials = run(oe, ve_p, vs_p, gt_p)
    return jnp.sum(partials)


# speedup vs baseline: 1.9350x; 1.9350x over previous
import jax
import jax.numpy as jnp
from jax import lax
from jax.experimental import pallas as pl
from jax.experimental.pallas import tpu as pltpu
from jax.experimental.pallas import tpu_sc as plsc

B, N, K, E = 4, 2500, 12, 7500
NW = 32
WPB = NW // B
NP = 2560
CH = NP // WPB
EP = 7680
GROUPS = CH // 16


def _sc_loss(oe_hbm, ve_hbm, vs_hbm, gt_hbm, out_hbm,
             oe_v, ve_v, vs_v, gt_v, loss_v, sem):
    c = lax.axis_index("c")
    s = lax.axis_index("s")
    wid = s * 2 + c
    batch = wid // WPB
    sub = lax.rem(wid, WPB)

    cp0 = pltpu.async_copy(oe_hbm.at[batch], oe_v, sem)
    cp1 = pltpu.async_copy(ve_hbm.at[batch, sub], ve_v, sem)
    cp2 = pltpu.async_copy(vs_hbm.at[batch, sub], vs_v, sem)
    cp3 = pltpu.async_copy(gt_hbm.at[batch, sub], gt_v, sem)
    cp0.wait()
    cp1.wait()
    cp2.wait()
    cp3.wait()

    def group(g, acc):
        jb = g * 16
        osum = plsc.load_gather(oe_v, [ve_v[0, pl.ds(jb, 16)]])
        for k in range(1, K):
            osum = osum + plsc.load_gather(oe_v, [ve_v[k, pl.ds(jb, 16)]])
        off = osum * (1.0 / K)
        d0 = vs_v[0, pl.ds(jb, 16)] - gt_v[0, pl.ds(jb, 16)]
        d1 = vs_v[1, pl.ds(jb, 16)] - gt_v[1, pl.ds(jb, 16)]
        d2 = vs_v[2, pl.ds(jb, 16)] - gt_v[2, pl.ds(jb, 16)]
        a = (d0 * d0 + d1 * d1 + d2 * d2) * (1.0 / 3.0)
        bd = (d0 + d1 + d2) * (2.0 / 3.0)
        return acc + a + off * (bd + off)

    loss16 = lax.fori_loop(0, GROUPS, group, jnp.zeros((16,), jnp.float32))
    loss_v[...] = loss16
    pltpu.sync_copy(loss_v, out_hbm.at[wid])


@jax.jit
def kernel(out_edges, gt_vs, vs, ve):
    oe = jnp.pad(out_edges[:, 0, :], ((0, 0), (0, EP - E)))
    ve_t = jnp.pad(ve.transpose(0, 2, 1), ((0, 0), (0, 0), (0, NP - N)),
                   constant_values=E)
    ve_r = ve_t.reshape(B, K, WPB, CH).transpose(0, 2, 1, 3)
    vs_t = jnp.pad(vs.transpose(0, 2, 1), ((0, 0), (0, 0), (0, NP - N)))
    vs_r = vs_t.reshape(B, 3, WPB, CH).transpose(0, 2, 1, 3)
    gt_t = jnp.pad(gt_vs.transpose(0, 2, 1), ((0, 0), (0, 0), (0, NP - N)))
    gt_r = gt_t.reshape(B, 3, WPB, CH).transpose(0, 2, 1, 3)

    mesh = plsc.VectorSubcoreMesh(core_axis_name="c", subcore_axis_name="s")
    run = pl.kernel(
        _sc_loss,
        out_type=jax.ShapeDtypeStruct((NW, 16), jnp.float32),
        mesh=mesh,
        compiler_params=pltpu.CompilerParams(needs_layout_passes=False),
        scratch_types=[
            pltpu.VMEM((EP,), jnp.float32),
            pltpu.VMEM((K, CH), jnp.int32),
            pltpu.VMEM((3, CH), jnp.float32),
            pltpu.VMEM((3, CH), jnp.float32),
            pltpu.VMEM((16,), jnp.float32),
            pltpu.SemaphoreType.DMA,
        ],
    )
    partials = run(oe, ve_r, vs_r, gt_r)
    return jnp.sum(partials)
